# baseline (speedup 1.0000x reference)
import functools

import jax
import jax.numpy as jnp
from jax import lax
from jax.experimental import pallas as pl
from jax.experimental.pallas import tpu as pltpu
from jax.experimental.pallas import tpu_sc as plsc

LATENT = 64
HIDDEN = 32


def _make_sc_gather(n_pair_rows, d2, b):
    info = plsc.get_sparse_core_info()
    nc, ns = info.num_cores, info.num_subcores
    nw = nc * ns
    assert b % (8 * nw) == 0 and d2 % info.num_lanes == 0
    b_per_w = b // nw
    mesh = plsc.VectorSubcoreMesh(core_axis_name="c", subcore_axis_name="s")

    @functools.partial(
        pl.kernel,
        mesh=mesh,
        out_type=jax.ShapeDtypeStruct((b, d2), jnp.float32),
        scratch_types=[
            pltpu.VMEM((b_per_w,), jnp.int32),
            pltpu.VMEM((b_per_w, d2), jnp.float32),
            pltpu.SemaphoreType.DMA,
        ],
    )
    def gather(table_hbm, idx_hbm, out_hbm, idx_v, rows_v, sem):
        wid = lax.axis_index("s") * nc + lax.axis_index("c")
        base = wid * b_per_w
        pltpu.sync_copy(idx_hbm.at[pl.ds(base, b_per_w)], idx_v)
        pltpu.async_copy(table_hbm.at[idx_v], rows_v, sem).wait()
        pltpu.sync_copy(rows_v, out_hbm.at[pl.ds(base, b_per_w)])

    return gather


def _tc_body(e2_ref, par_ref, w_ref, bias_ref, out_ref):
    e2 = e2_ref[...]
    e = jnp.where(par_ref[...] == 1, e2[:, LATENT:], e2[:, :LATENT])
    h = lax.dot_general(
        e, w_ref[...], (((1,), (1,)), ((), ())),
        preferred_element_type=jnp.float32,
    ) + bias_ref[...]
    out_ref[...] = jnp.sum(h * h, axis=1, keepdims=True)


def _make_tc_mlp(batch, blk):
    grid = (batch // blk,)
    return pl.pallas_call(
        _tc_body,
        grid=grid,
        in_specs=[
            pl.BlockSpec((blk, 2 * LATENT), lambda i: (i, 0)),
            pl.BlockSpec((blk, 1), lambda i: (i, 0)),
            pl.BlockSpec((HIDDEN, LATENT), lambda i: (0, 0)),
            pl.BlockSpec((1, HIDDEN), lambda i: (0, 0)),
        ],
        out_specs=pl.BlockSpec((blk, 1), lambda i: (i, 0)),
        out_shape=jax.ShapeDtypeStruct((batch, 1), jnp.float32),
    )


def kernel(user_ids, item_ids, user_table, item_table, W_user, b_user,
           W_item, b_item):
    batch = user_ids.shape[0]
    ids = user_ids.astype(jnp.int32)
    table2 = user_table.reshape(user_table.shape[0] // 2, 2 * LATENT)
    emb2 = _make_sc_gather(table2.shape[0], 2 * LATENT, batch)(
        table2, ids >> 1)
    parity = (ids & 1).reshape(batch, 1)
    out = _make_tc_mlp(batch, 2048)(
        emb2, parity, W_user, b_user.reshape(1, HIDDEN))
    return out.reshape(batch)


# speedup vs baseline: 4.3690x; 4.3690x over previous
import functools

import jax
import jax.numpy as jnp
from jax import lax
from jax.experimental import pallas as pl
from jax.experimental.pallas import tpu as pltpu
from jax.experimental.pallas import tpu_sc as plsc

LATENT = 64
HIDDEN = 32
CW = 8192


def _tc_body(w_ref, bias_ref, et_ref, out_ref):
    h = lax.dot_general(
        w_ref[...], et_ref[...], (((1,), (0,)), ((), ())),
        preferred_element_type=jnp.float32,
    ) + bias_ref[...]
    out_ref[...] = jnp.sum(h * h, axis=0)


def _make_tc_dense(n_users, n_users_pad):
    grid = (pl.cdiv(n_users, CW),)
    return pl.pallas_call(
        _tc_body,
        grid=grid,
        in_specs=[
            pl.BlockSpec((HIDDEN, LATENT), lambda i: (0, 0)),
            pl.BlockSpec((HIDDEN, 1), lambda i: (0, 0)),
            pl.BlockSpec((LATENT, CW), lambda i: (0, i)),
        ],
        out_specs=pl.BlockSpec((CW,), lambda i: (i,)),
        out_shape=jax.ShapeDtypeStruct((n_users_pad,), jnp.float32),
    )


def _make_sc_lookup(n_s, b):
    info = plsc.get_sparse_core_info()
    nc, ns = info.num_cores, info.num_subcores
    nw = nc * ns
    assert b % (8 * nw) == 0
    b_per_w = b // nw
    mesh = plsc.VectorSubcoreMesh(core_axis_name="c", subcore_axis_name="s")

    @functools.partial(
        pl.kernel,
        mesh=mesh,
        out_type=jax.ShapeDtypeStruct((b,), jnp.float32),
        scratch_types=[
            pltpu.VMEM((b_per_w,), jnp.int32),
            pltpu.VMEM((b_per_w,), jnp.float32),
            pltpu.SemaphoreType.DMA,
        ],
    )
    def lookup(s_hbm, idx_hbm, out_hbm, idx_v, val_v, sem):
        wid = lax.axis_index("s") * nc + lax.axis_index("c")
        base = wid * b_per_w
        pltpu.sync_copy(idx_hbm.at[pl.ds(base, b_per_w)], idx_v)
        pltpu.async_copy(s_hbm.at[idx_v], val_v, sem).wait()
        pltpu.sync_copy(val_v, out_hbm.at[pl.ds(base, b_per_w)])

    return lookup


def kernel(user_ids, item_ids, user_table, item_table, W_user, b_user,
           W_item, b_item):
    batch = user_ids.shape[0]
    n_users = user_table.shape[0]
    n_users_pad = pl.cdiv(n_users, CW) * CW
    ids = user_ids.astype(jnp.int32)
    s = _make_tc_dense(n_users, n_users_pad)(
        W_user, b_user.reshape(HIDDEN, 1), user_table.T)
    return _make_sc_lookup(n_users_pad, batch)(s, ids)


# speedup vs baseline: 5.6689x; 1.2975x over previous
import functools

import jax
import jax.numpy as jnp
from jax import lax
from jax.experimental import pallas as pl
from jax.experimental.pallas import tpu as pltpu
from jax.experimental.pallas import tpu_sc as plsc

LATENT = 64
HIDDEN = 32
CW = 16384


def _tc_body(w_ref, bias_ref, et_ref, out_ref):
    h = lax.dot_general(
        w_ref[...], et_ref[...], (((1,), (0,)), ((), ())),
        preferred_element_type=jnp.float32,
    ) + bias_ref[...]
    out_ref[...] = jnp.sum(h * h, axis=0)


def _make_tc_dense(n_users, n_users_pad):
    grid = (pl.cdiv(n_users, CW),)
    return pl.pallas_call(
        _tc_body,
        grid=grid,
        in_specs=[
            pl.BlockSpec((HIDDEN, LATENT), lambda i: (0, 0)),
            pl.BlockSpec((HIDDEN, 1), lambda i: (0, 0)),
            pl.BlockSpec((LATENT, CW), lambda i: (0, i)),
        ],
        out_specs=pl.BlockSpec((CW,), lambda i: (i,)),
        out_shape=jax.ShapeDtypeStruct((n_users_pad,), jnp.float32),
    )


def _make_sc_lookup(n_s, b):
    info = plsc.get_sparse_core_info()
    nc, ns = info.num_cores, info.num_subcores
    nw = nc * ns
    assert b % (8 * nw) == 0
    b_per_w = b // nw
    mesh = plsc.VectorSubcoreMesh(core_axis_name="c", subcore_axis_name="s")

    @functools.partial(
        pl.kernel,
        mesh=mesh,
        out_type=jax.ShapeDtypeStruct((b,), jnp.float32),
        scratch_types=[
            pltpu.VMEM((b_per_w,), jnp.int32),
            pltpu.VMEM((b_per_w,), jnp.float32),
            pltpu.SemaphoreType.DMA,
        ],
    )
    def lookup(s_hbm, idx_hbm, out_hbm, idx_v, val_v, sem):
        wid = lax.axis_index("s") * nc + lax.axis_index("c")
        base = wid * b_per_w
        pltpu.sync_copy(idx_hbm.at[pl.ds(base, b_per_w)], idx_v)
        pltpu.async_copy(s_hbm.at[idx_v], val_v, sem).wait()
        pltpu.sync_copy(val_v, out_hbm.at[pl.ds(base, b_per_w)])

    return lookup


def kernel(user_ids, item_ids, user_table, item_table, W_user, b_user,
           W_item, b_item):
    batch = user_ids.shape[0]
    n_users = user_table.shape[0]
    n_users_pad = pl.cdiv(n_users, CW) * CW
    ids = user_ids.astype(jnp.int32)
    s = _make_tc_dense(n_users, n_users_pad)(
        W_user, b_user.reshape(HIDDEN, 1), user_table.T)
    return _make_sc_lookup(n_users_pad, batch)(s, ids)


# speedup vs baseline: 6.6365x; 1.1707x over previous
import functools

import jax
import jax.numpy as jnp
from jax import lax
from jax.experimental import pallas as pl
from jax.experimental.pallas import tpu as pltpu
from jax.experimental.pallas import tpu_sc as plsc

LATENT = 64
HIDDEN = 32
CW = 32768


def _tc_body(w_ref, bias_ref, et_ref, out_ref):
    h = lax.dot_general(
        w_ref[...], et_ref[...], (((1,), (0,)), ((), ())),
        preferred_element_type=jnp.float32,
    ) + bias_ref[...]
    out_ref[...] = jnp.sum(h * h, axis=0)


def _make_tc_dense(n_users, n_users_pad):
    grid = (pl.cdiv(n_users, CW),)
    return pl.pallas_call(
        _tc_body,
        grid=grid,
        in_specs=[
            pl.BlockSpec((HIDDEN, LATENT), lambda i: (0, 0)),
            pl.BlockSpec((HIDDEN, 1), lambda i: (0, 0)),
            pl.BlockSpec((LATENT, CW), lambda i: (0, i)),
        ],
        out_specs=pl.BlockSpec((CW,), lambda i: (i,)),
        out_shape=jax.ShapeDtypeStruct((n_users_pad,), jnp.float32),
    )


def _make_sc_lookup(n_s, b):
    info = plsc.get_sparse_core_info()
    nc, ns = info.num_cores, info.num_subcores
    nw = nc * ns
    assert b % (8 * nw) == 0
    b_per_w = b // nw
    mesh = plsc.VectorSubcoreMesh(core_axis_name="c", subcore_axis_name="s")

    @functools.partial(
        pl.kernel,
        mesh=mesh,
        out_type=jax.ShapeDtypeStruct((b,), jnp.float32),
        scratch_types=[
            pltpu.VMEM((b_per_w,), jnp.int32),
            pltpu.VMEM((b_per_w,), jnp.float32),
            pltpu.SemaphoreType.DMA,
        ],
    )
    def lookup(s_hbm, idx_hbm, out_hbm, idx_v, val_v, sem):
        wid = lax.axis_index("s") * nc + lax.axis_index("c")
        base = wid * b_per_w
        pltpu.sync_copy(idx_hbm.at[pl.ds(base, b_per_w)], idx_v)
        pltpu.async_copy(s_hbm.at[idx_v], val_v, sem).wait()
        pltpu.sync_copy(val_v, out_hbm.at[pl.ds(base, b_per_w)])

    return lookup


def kernel(user_ids, item_ids, user_table, item_table, W_user, b_user,
           W_item, b_item):
    batch = user_ids.shape[0]
    n_users = user_table.shape[0]
    n_users_pad = pl.cdiv(n_users, CW) * CW
    ids = user_ids.astype(jnp.int32)
    s = _make_tc_dense(n_users, n_users_pad)(
        W_user, b_user.reshape(HIDDEN, 1), user_table.T)
    return _make_sc_lookup(n_users_pad, batch)(s, ids)
